# baseline (speedup 1.0000x reference)
import functools

import jax
import jax.numpy as jnp
from jax import lax
from jax.experimental import pallas as pl
from jax.experimental.pallas import tpu as pltpu

B, C = 16384, 100
M = 1000000
BLK_B = 2048
GRID = B // BLK_B
BL = 16384
NBLK = (M + BL - 1) // BL


def _dense_body(ow_ref, os_ref, tgt_ref, rev_ref, loss_ref, acc_ref):
    i = pl.program_id(0)

    w = ow_ref[...]
    s = os_ref[...]
    t = tgt_ref[...]

    mw = jnp.max(w, axis=0, keepdims=True)
    ew = jnp.exp(w - mw)
    sw = jnp.sum(ew, axis=0, keepdims=True)
    pw = ew / sw
    logpw = (w - mw) - jnp.log(sw)

    ms = jnp.max(s, axis=0, keepdims=True)
    es = jnp.exp(s - ms)
    ss = jnp.sum(es, axis=0, keepdims=True)
    ps = es / ss
    logps = (s - ms) - jnp.log(ss)

    pos = t > 0.0
    neg = jnp.where(pos, 0.0, 1.0)
    sup = neg * (-jnp.log(jnp.abs(1.0 - pw) + 1e-9)
                 - jnp.log(jnp.abs(1.0 - ps) + 1e-9))
    kl = 2.0 * jnp.where(pos, t * jnp.log(jnp.where(pos, t, 1.0)), 0.0) \
        - t * (logpw + logps)
    partial = jnp.sum(sup) + jnp.sum(kl)

    g = jnp.where(pos, jnp.sqrt(pw * ps), 0.0)
    rev_ref[...] = g / (jnp.sum(g, axis=0, keepdims=True) + 1e-9)

    @pl.when(i == 0)
    def _():
        acc_ref[0] = 0.0

    acc_ref[0] += partial

    @pl.when(i == GRID - 1)
    def _():
        loss_ref[0, 0] = acc_ref[0] * (1.0 / B)


_dense = pl.pallas_call(
    _dense_body,
    grid=(GRID,),
    in_specs=[
        pl.BlockSpec((C, BLK_B), lambda i: (0, i)),
        pl.BlockSpec((C, BLK_B), lambda i: (0, i)),
        pl.BlockSpec((C, BLK_B), lambda i: (0, i)),
    ],
    out_specs=[
        pl.BlockSpec((C, BLK_B), lambda i: (0, i)),
        pl.BlockSpec((1, 1), lambda i: (0, 0), memory_space=pltpu.SMEM),
    ],
    out_shape=[
        jax.ShapeDtypeStruct((C, B), jnp.float32),
        jax.ShapeDtypeStruct((1, 1), jnp.float32),
    ],
    scratch_shapes=[pltpu.SMEM((1,), jnp.float32)],
)


def _apply_body(starts_ref, sidx_ref, osrc_ref, conf_ref, rev_ref, out_ref):
    i = pl.program_id(0)
    out_ref[...] = conf_ref[...]
    base = i * BL

    def body(k, _):
        l = sidx_ref[k] - base
        o = osrc_ref[k]
        l0 = pl.multiple_of((l // 128) * 128, 128)
        o0 = pl.multiple_of((o // 128) * 128, 128)
        lane = lax.broadcasted_iota(jnp.int32, (1, 128), 1)
        rwin = rev_ref[:, pl.ds(o0, 128)]
        col = jnp.sum(jnp.where(lane == (o - o0), rwin, 0.0),
                      axis=1, keepdims=True)
        owin = out_ref[:, pl.ds(l0, 128)]
        out_ref[:, pl.ds(l0, 128)] = jnp.where(lane == (l - l0), col, owin)
        return 0

    lax.fori_loop(starts_ref[i], starts_ref[i + 1], body, 0)


_apply = pl.pallas_call(
    _apply_body,
    grid_spec=pltpu.PrefetchScalarGridSpec(
        num_scalar_prefetch=3,
        grid=(NBLK,),
        in_specs=[
            pl.BlockSpec((C, BL), lambda i, *_: (0, i)),
            pl.BlockSpec((C, B), lambda i, *_: (0, 0)),
        ],
        out_specs=pl.BlockSpec((C, BL), lambda i, *_: (0, i)),
    ),
    out_shape=jax.ShapeDtypeStruct((C, M), jnp.float32),
)


def kernel(output_w, output_s, index, confidence):
    index = index.astype(jnp.int32)
    target = jnp.take(confidence, index, axis=0)
    rev_t, loss = _dense(output_w.T, output_s.T, target.T)

    order = jnp.argsort(index).astype(jnp.int32)
    sidx = jnp.take(index, order)
    pos = jnp.arange(B, dtype=jnp.int32)
    is_last = jnp.concatenate(
        [sidx[:-1] != sidx[1:], jnp.ones((1,), dtype=bool)])
    lastpos = lax.cummin(jnp.where(is_last, pos, B - 1), reverse=True)
    osrc = jnp.take(order, lastpos)
    starts = jnp.searchsorted(
        sidx, jnp.arange(0, (NBLK + 1) * BL, BL, dtype=jnp.int32)
    ).astype(jnp.int32)

    out_t = _apply(starts, sidx, osrc, confidence.T, rev_t)
    return loss[0, 0], out_t.T


# speedup vs baseline: 3.3678x; 3.3678x over previous
import functools

import jax
import jax.numpy as jnp
from jax import lax
from jax.experimental import pallas as pl
from jax.experimental.pallas import tpu as pltpu

B, C = 16384, 100
M = 1000000
BLK_B = 2048
GRID = B // BLK_B


def _dense_body(ow_ref, os_ref, tgt_ref, rev_ref, loss_ref, acc_ref):
    i = pl.program_id(0)

    w = ow_ref[...]
    s = os_ref[...]
    t = tgt_ref[...]

    mw = jnp.max(w, axis=1, keepdims=True)
    ew = jnp.exp(w - mw)
    sw = jnp.sum(ew, axis=1, keepdims=True)
    pw = ew / sw
    logpw = (w - mw) - jnp.log(sw)

    ms = jnp.max(s, axis=1, keepdims=True)
    es = jnp.exp(s - ms)
    ss = jnp.sum(es, axis=1, keepdims=True)
    ps = es / ss
    logps = (s - ms) - jnp.log(ss)

    pos = t > 0.0
    neg = jnp.where(pos, 0.0, 1.0)
    sup = neg * (-jnp.log(jnp.abs(1.0 - pw) + 1e-9)
                 - jnp.log(jnp.abs(1.0 - ps) + 1e-9))
    kl = 2.0 * jnp.where(pos, t * jnp.log(jnp.where(pos, t, 1.0)), 0.0) \
        - t * (logpw + logps)
    partial = jnp.sum(sup) + jnp.sum(kl)

    g = jnp.where(pos, jnp.sqrt(pw * ps), 0.0)
    rev_ref[...] = g / (jnp.sum(g, axis=1, keepdims=True) + 1e-9)

    @pl.when(i == 0)
    def _():
        acc_ref[0] = 0.0

    acc_ref[0] += partial

    @pl.when(i == GRID - 1)
    def _():
        loss_ref[0, 0] = acc_ref[0] * (1.0 / B)


_dense = pl.pallas_call(
    _dense_body,
    grid=(GRID,),
    in_specs=[
        pl.BlockSpec((BLK_B, C), lambda i: (i, 0)),
        pl.BlockSpec((BLK_B, C), lambda i: (i, 0)),
        pl.BlockSpec((BLK_B, C), lambda i: (i, 0)),
    ],
    out_specs=[
        pl.BlockSpec((BLK_B, C), lambda i: (i, 0)),
        pl.BlockSpec((1, 1), lambda i: (0, 0), memory_space=pltpu.SMEM),
    ],
    out_shape=[
        jax.ShapeDtypeStruct((B, C), jnp.float32),
        jax.ShapeDtypeStruct((1, 1), jnp.float32),
    ],
    scratch_shapes=[pltpu.SMEM((1,), jnp.float32)],
)


def _apply_body(sidx_ref, osrc_ref, conf_ref, rev_ref, out_ref, sem):
    del conf_ref

    def issue(k, _):
        pltpu.make_async_copy(
            rev_ref.at[pl.ds(osrc_ref[k], 1), :],
            out_ref.at[pl.ds(sidx_ref[k], 1), :],
            sem,
        ).start()
        return 0

    lax.fori_loop(0, B, issue, 0, unroll=8)

    pltpu.make_async_copy(
        rev_ref.at[pl.ds(0, B), :],
        out_ref.at[pl.ds(0, B), :],
        sem,
    ).wait()


_apply = pl.pallas_call(
    _apply_body,
    grid_spec=pltpu.PrefetchScalarGridSpec(
        num_scalar_prefetch=2,
        grid=(1,),
        in_specs=[
            pl.BlockSpec(memory_space=pl.ANY),
            pl.BlockSpec((B, C), lambda i, *_: (0, 0)),
        ],
        out_specs=pl.BlockSpec(memory_space=pl.ANY),
        scratch_shapes=[pltpu.SemaphoreType.DMA],
    ),
    out_shape=jax.ShapeDtypeStruct((M, C), jnp.float32),
    input_output_aliases={2: 0},
)


def kernel(output_w, output_s, index, confidence):
    index = index.astype(jnp.int32)
    target = jnp.take(confidence, index, axis=0)
    rev, loss = _dense(output_w, output_s, target)

    order = jnp.argsort(index).astype(jnp.int32)
    sidx = jnp.take(index, order)
    pos = jnp.arange(B, dtype=jnp.int32)
    is_last = jnp.concatenate(
        [sidx[:-1] != sidx[1:], jnp.ones((1,), dtype=bool)])
    lastpos = lax.cummin(jnp.where(is_last, pos, B - 1), reverse=True)
    osrc = jnp.take(order, lastpos)

    new_confidence = _apply(sidx, osrc, confidence, rev)
    return loss[0, 0], new_confidence
